# baseline (speedup 1.0000x reference)
import jax
import jax.numpy as jnp
from jax import lax
from jax.experimental import pallas as pl
from jax.experimental.pallas import tpu as pltpu


def _round_up(x, m):
    return (x + m - 1) // m * m


def _combine_kernel(active_ref, remap_ref,
                    x_ref, gates_ref,
                    w1_ref, b1_ref, w2_ref,
                    binit_ref,
                    out_ref):
    del remap_ref
    e = pl.program_id(1)
    hc = pl.program_id(2)

    @pl.when(jnp.logical_and(e == 0, hc == 0))
    def _init():
        out_ref[...] = binit_ref[...].astype(jnp.float32)

    @pl.when(active_ref[e] != 0)
    def _compute():
        col = lax.broadcasted_iota(jnp.int32, gates_ref.shape, 1)
        gate = jnp.sum(jnp.where(col == e, gates_ref[...], 0.0),
                       axis=1, keepdims=True)

        h = jnp.dot(x_ref[...], w1_ref[...],
                    preferred_element_type=jnp.float32) + b1_ref[...]
        h = jax.nn.gelu(h, approximate=True) * gate
        y = jnp.dot(h.astype(jnp.bfloat16), w2_ref[...],
                    preferred_element_type=jnp.float32)
        out_ref[...] += y


def _moe_combine(x, gates_te, w1, b1, w2, b2, active, nondegenerate, out_dtype):
    T, D = x.shape
    E, _, H = w1.shape

    xc = x.astype(jnp.bfloat16)
    w1c = w1.astype(jnp.bfloat16)
    w2c = w2.astype(jnp.bfloat16)
    b1f = b1.astype(jnp.float32)
    gates_te = gates_te.astype(jnp.float32)
    active = active.astype(jnp.int32)

    binit = ((gates_te @ b2.reshape(E, D).astype(jnp.float32))
             * nondegenerate.astype(jnp.float32)).astype(jnp.bfloat16)

    tile_t = _round_up(pl.cdiv(_round_up(T, 8), 2), 8) if T >= 16 else _round_up(T, 8)
    t_pad = _round_up(T, tile_t)
    if t_pad != T:
        xc = jnp.pad(xc, ((0, t_pad - T), (0, 0)))
        gates_te = jnp.pad(gates_te, ((0, t_pad - T), (0, 0)))
        binit = jnp.pad(binit, ((0, t_pad - T), (0, 0)))
    num_tiles = t_pad // tile_t

    tile_h = 1024 if (H % 1024 == 0 and H > 1024) else H
    n_hc = H // tile_h

    idx = jnp.arange(E, dtype=jnp.int32)
    run_max = lax.cummax(jnp.where(active > 0, idx, -1))
    first_active = jnp.where(jnp.any(active > 0),
                             jnp.argmax(active > 0).astype(jnp.int32),
                             jnp.int32(0))
    remap = jnp.where(run_max < 0, first_active, run_max).astype(jnp.int32)

    cost = pl.CostEstimate(
        flops=int(4 * t_pad * E * D * H),
        transcendentals=int(t_pad * E * H),
        bytes_accessed=int(t_pad * D * (2 + 4 + 4) + t_pad * E * 4
                           + num_tiles * E * (2 * D * H * 2 + (H + D) * 4)),
    )

    grid_spec = pltpu.PrefetchScalarGridSpec(
        num_scalar_prefetch=2,
        grid=(num_tiles, E, n_hc),
        in_specs=[
            pl.BlockSpec((tile_t, D), lambda t, e, hc, act, rmp: (t, 0)),
            pl.BlockSpec((tile_t, E), lambda t, e, hc, act, rmp: (t, 0)),
            pl.BlockSpec((None, D, tile_h),
                         lambda t, e, hc, act, rmp: (rmp[e], 0, hc)),
            pl.BlockSpec((None, 1, tile_h),
                         lambda t, e, hc, act, rmp: (rmp[e], 0, hc)),
            pl.BlockSpec((None, tile_h, D),
                         lambda t, e, hc, act, rmp: (rmp[e], hc, 0)),
            pl.BlockSpec((tile_t, D), lambda t, e, hc, act, rmp: (t, 0)),
        ],
        out_specs=pl.BlockSpec((tile_t, D), lambda t, e, hc, act, rmp: (t, 0)),
    )
    out = pl.pallas_call(
        _combine_kernel,
        out_shape=jax.ShapeDtypeStruct((t_pad, D), jnp.float32),
        grid_spec=grid_spec,
        compiler_params=pltpu.CompilerParams(
            dimension_semantics=("parallel", "arbitrary", "arbitrary"),
            vmem_limit_bytes=64 * 1024 * 1024),
        cost_estimate=cost,
        name="moe_combine",
    )(active, remap, xc, gates_te, w1c, b1f, w2c, binit)

    return out[:T].astype(out_dtype)


def kernel(inputs, mask, wr, w1, b1, w2, b2):
    B, S, D = inputs.shape
    x = inputs.reshape(-1, D)
    T = x.shape[0]
    E = wr.shape[1]

    logits = (x.astype(jnp.float32) @ wr.astype(jnp.float32)) \
        * mask.astype(jnp.float32)[None, :]
    sum_of_logits = jnp.sum(logits)

    logits_full = jnp.concatenate(
        [logits, jnp.zeros((T, 1), logits.dtype)], axis=1)

    all_probs = jax.nn.softmax(logits_full, axis=1)
    weights, selected_experts = lax.top_k(all_probs, 2)

    onehot = jax.nn.one_hot(selected_experts, E + 1, dtype=weights.dtype)
    gates = jnp.sum(weights[:, :, None] * onehot, axis=1)[:, :E]

    nondegenerate = sum_of_logits >= 1e-20
    active = jnp.sum(onehot[..., :E], axis=(0, 1)) > 0
    active = jnp.logical_and(active, nondegenerate).astype(jnp.int32)

    results = _moe_combine(x, gates, w1, b1, w2, b2, active, nondegenerate,
                           inputs.dtype)

    aux = {"router_logits": logits_full, "selected_experts": selected_experts}
    return results.reshape(inputs.shape), aux
